# baseline (speedup 1.0000x reference)
import functools

import jax
import jax.numpy as jnp
from jax import lax
from jax.experimental import pallas as pl
from jax.experimental.pallas import tpu as pltpu
from jax.experimental.pallas import tpu_sc as plsc

_TOP_K = 2
_EXPERTS = 16
_BLK = 1024


def _logits_t_kernel(x_ref, w_ref, b_ref, out_ref):
    out_ref[...] = lax.dot_general(
        w_ref[...], x_ref[...],
        (((1,), (1,)), ((), ())),
        preferred_element_type=jnp.float32,
    ) + b_ref[...]


_SC_INFO = plsc.get_sparse_core_info()
_NW = _SC_INFO.num_cores * _SC_INFO.num_subcores
_LANES = _SC_INFO.num_lanes


def _route_sc_body(tpw, logits_hbm, out_hbm, idx_hbm, lbuf, obuf, ibuf):
    wid = lax.axis_index("s") * _SC_INFO.num_cores + lax.axis_index("c")
    base = wid * tpw
    pltpu.sync_copy(logits_hbm.at[:, pl.ds(base, tpw)], lbuf)

    neg_inf = jnp.full((_LANES,), -jnp.inf, dtype=jnp.float32)
    big = jnp.full((_LANES,), _EXPERTS, dtype=jnp.int32)
    zero = jnp.zeros((_LANES,), dtype=jnp.float32)

    for g in range(tpw // _LANES):
        sl = pl.ds(g * _LANES, _LANES)
        cols = [lbuf[e, sl] for e in range(_EXPERTS)]

        v1 = cols[0]
        for e in range(1, _EXPERTS):
            v1 = jnp.maximum(v1, cols[e])
        idx1 = big
        for e in range(_EXPERTS):
            cand = jnp.full((_LANES,), e, dtype=jnp.int32)
            idx1 = jnp.minimum(idx1, jnp.where(cols[e] == v1, cand, big))

        cols2 = [jnp.where(idx1 == e, neg_inf, cols[e])
                 for e in range(_EXPERTS)]
        v2 = cols2[0]
        for e in range(1, _EXPERTS):
            v2 = jnp.maximum(v2, cols2[e])
        idx2 = big
        for e in range(_EXPERTS):
            cand = jnp.full((_LANES,), e, dtype=jnp.int32)
            idx2 = jnp.minimum(idx2, jnp.where(cols2[e] == v2, cand, big))

        t = jnp.exp(v2 - v1)
        denom = 1.0 + t
        p1 = 1.0 / denom
        p2 = t / denom

        for e in range(_EXPERTS):
            obuf[e, sl] = (jnp.where(idx1 == e, p1, zero)
                           + jnp.where(idx2 == e, p2, zero))
        ibuf[0, sl] = idx1
        ibuf[1, sl] = idx2

    pltpu.sync_copy(obuf, out_hbm.at[:, pl.ds(base, tpw)])
    pltpu.sync_copy(ibuf, idx_hbm.at[:, pl.ds(base, tpw)])


def _finalize_kernel(outt_ref, idxt_ref, out_ref, idx_ref):
    out_ref[...] = outt_ref[...].T
    idx_ref[...] = idxt_ref[...].T


def kernel(x, route_W, route_b, noise_W, noise_b):
    del noise_W, noise_b
    tokens = x.shape[0]
    bcol = route_b.reshape(_EXPERTS, 1)
    grid = (tokens // _BLK,)
    logits_t = pl.pallas_call(
        _logits_t_kernel,
        grid=grid,
        in_specs=[
            pl.BlockSpec((_BLK, x.shape[1]), lambda i: (i, 0)),
            pl.BlockSpec((_EXPERTS, x.shape[1]), lambda i: (0, 0)),
            pl.BlockSpec((_EXPERTS, 1), lambda i: (0, 0)),
        ],
        out_specs=pl.BlockSpec((_EXPERTS, _BLK), lambda i: (0, i)),
        out_shape=jax.ShapeDtypeStruct((_EXPERTS, tokens), jnp.float32),
        compiler_params=pltpu.CompilerParams(
            dimension_semantics=("arbitrary",),
        ),
    )(x, route_W, bcol)

    tpw = tokens // _NW
    mesh = plsc.VectorSubcoreMesh(core_axis_name="c", subcore_axis_name="s")
    route = functools.partial(
        pl.kernel,
        out_type=[
            jax.ShapeDtypeStruct((_EXPERTS, tokens), jnp.float32),
            jax.ShapeDtypeStruct((_TOP_K, tokens), jnp.int32),
        ],
        mesh=mesh,
        scratch_types=[
            pltpu.VMEM((_EXPERTS, tpw), jnp.float32),
            pltpu.VMEM((_EXPERTS, tpw), jnp.float32),
            pltpu.VMEM((_TOP_K, tpw), jnp.int32),
        ],
    )(functools.partial(_route_sc_body, tpw))
    out_t, idx_t = route(logits_t)

    out, idx = pl.pallas_call(
        _finalize_kernel,
        grid=grid,
        in_specs=[
            pl.BlockSpec((_EXPERTS, _BLK), lambda i: (0, i)),
            pl.BlockSpec((_TOP_K, _BLK), lambda i: (0, i)),
        ],
        out_specs=[
            pl.BlockSpec((_BLK, _EXPERTS), lambda i: (i, 0)),
            pl.BlockSpec((_BLK, _TOP_K), lambda i: (i, 0)),
        ],
        out_shape=[
            jax.ShapeDtypeStruct((tokens, _EXPERTS), jnp.float32),
            jax.ShapeDtypeStruct((tokens, _TOP_K), jnp.int32),
        ],
        compiler_params=pltpu.CompilerParams(
            dimension_semantics=("arbitrary",),
        ),
    )(out_t, idx_t)
    return (out, idx)
